# baseline (speedup 1.0000x reference)
import functools

import jax
import jax.numpy as jnp
from jax import lax
from jax.experimental import pallas as pl
from jax.experimental.pallas import tpu as pltpu
from jax.experimental.pallas import tpu_sc as plsc

_NC = 2
_NS = 16
_NW = _NC * _NS


@functools.lru_cache(maxsize=None)
def _make_sc(seq_len, batch, embed_dim, dtype):
    rows_per_w = seq_len // _NW
    chunk = min(rows_per_w, 32)
    n_chunks = rows_per_w // chunk
    n_buf = 4
    mesh = plsc.VectorSubcoreMesh(core_axis_name="c", subcore_axis_name="s")

    @functools.partial(
        pl.kernel,
        mesh=mesh,
        out_type=jax.ShapeDtypeStruct((seq_len, batch, embed_dim), dtype),
        scratch_types=(
            [pltpu.VMEM((chunk, embed_dim), dtype) for _ in range(n_buf)]
            + [pltpu.SemaphoreType.DMA for _ in range(n_buf)]
            + [pltpu.SemaphoreType.DMA for _ in range(n_buf)]
        ),
    )
    def k(table_hbm, out_hbm, *scratch):
        bufs = scratch[:n_buf]
        lsem = scratch[n_buf : 2 * n_buf]
        wsem = scratch[2 * n_buf : 3 * n_buf]
        wid = lax.axis_index("s") * _NC + lax.axis_index("c")
        base = wid * rows_per_w

        def load(c):
            s0 = base + c * chunk
            d = pltpu.make_async_copy(
                table_hbm.at[pl.ds(s0, chunk)], bufs[c % n_buf], lsem[c % n_buf]
            )
            d.start()
            return d

        def writes(c):
            s0 = base + c * chunk
            ds = []
            for b in range(batch):
                d = pltpu.make_async_copy(
                    bufs[c % n_buf],
                    out_hbm.at[pl.ds(s0, chunk), b],
                    wsem[c % n_buf],
                )
                d.start()
                ds.append(d)
            return ds

        depth = n_buf - 1
        pending_w = [None] * n_buf
        lds = {}

        def prefetch(c):
            if c < n_chunks:
                nb = c % n_buf
                if pending_w[nb] is not None:
                    for d in pending_w[nb]:
                        d.wait()
                    pending_w[nb] = None
                lds[c] = load(c)

        for c in range(min(depth, n_chunks)):
            lds[c] = load(c)
        for c in range(n_chunks):
            prefetch(c + depth)
            lds.pop(c).wait()
            pending_w[c % n_buf] = writes(c)
        for ds in pending_w:
            if ds is not None:
                for d in ds:
                    d.wait()

    return k


def kernel(x, pos_embedding):
    seq_len, batch = x.shape
    max_len, embed_dim = pos_embedding.shape
    k = _make_sc(seq_len, batch, embed_dim, pos_embedding.dtype)
    return k(pos_embedding)


# speedup vs baseline: 1.0014x; 1.0014x over previous
import functools

import jax
import jax.numpy as jnp
from jax import lax
from jax.experimental import pallas as pl
from jax.experimental.pallas import tpu as pltpu
from jax.experimental.pallas import tpu_sc as plsc

_NC = 2
_NS = 16
_NW = _NC * _NS


@functools.lru_cache(maxsize=None)
def _make_sc(seq_len, batch, embed_dim, dtype):
    rows_per_w = seq_len // _NW
    chunk = min(rows_per_w, 64)
    n_chunks = rows_per_w // chunk
    n_buf = 2
    mesh = plsc.VectorSubcoreMesh(core_axis_name="c", subcore_axis_name="s")

    @functools.partial(
        pl.kernel,
        mesh=mesh,
        out_type=jax.ShapeDtypeStruct((seq_len, batch, embed_dim), dtype),
        scratch_types=(
            [pltpu.VMEM((chunk, embed_dim), dtype) for _ in range(n_buf)]
            + [pltpu.SemaphoreType.DMA for _ in range(n_buf)]
            + [pltpu.SemaphoreType.DMA for _ in range(n_buf)]
        ),
    )
    def k(table_hbm, out_hbm, *scratch):
        bufs = scratch[:n_buf]
        lsem = scratch[n_buf : 2 * n_buf]
        wsem = scratch[2 * n_buf : 3 * n_buf]
        wid = lax.axis_index("s") * _NC + lax.axis_index("c")
        base = wid * rows_per_w

        def load(c):
            s0 = base + c * chunk
            d = pltpu.make_async_copy(
                table_hbm.at[pl.ds(s0, chunk)], bufs[c % n_buf], lsem[c % n_buf]
            )
            d.start()
            return d

        def writes(c):
            s0 = base + c * chunk
            ds = []
            for b in range(batch):
                d = pltpu.make_async_copy(
                    bufs[c % n_buf],
                    out_hbm.at[pl.ds(s0, chunk), b],
                    wsem[c % n_buf],
                )
                d.start()
                ds.append(d)
            return ds

        depth = n_buf - 1
        pending_w = [None] * n_buf
        lds = {}

        for c in range(min(depth, n_chunks)):
            lds[c] = load(c)
        for c in range(n_chunks):
            lds.pop(c).wait()
            pending_w[c % n_buf] = writes(c)
            nxt = c + depth
            if nxt < n_chunks:
                nb = nxt % n_buf
                if pending_w[nb] is not None:
                    for d in pending_w[nb]:
                        d.wait()
                    pending_w[nb] = None
                lds[nxt] = load(nxt)
        for ds in pending_w:
            if ds is not None:
                for d in ds:
                    d.wait()

    return k


def kernel(x, pos_embedding):
    seq_len, batch = x.shape
    max_len, embed_dim = pos_embedding.shape
    k = _make_sc(seq_len, batch, embed_dim, pos_embedding.dtype)
    return k(pos_embedding)


# speedup vs baseline: 1.0679x; 1.0663x over previous
import functools

import jax
import jax.numpy as jnp
from jax import lax
from jax.experimental import pallas as pl
from jax.experimental.pallas import tpu as pltpu
from jax.experimental.pallas import tpu_sc as plsc

_NC = 2
_NS = 16
_NW = _NC * _NS


@functools.lru_cache(maxsize=None)
def _make_sc(seq_len, batch, embed_dim, dtype):
    rows_per_w = seq_len // _NW
    chunk = min(rows_per_w, 64)
    n_chunks = rows_per_w // chunk
    n_buf = 2
    mesh = plsc.VectorSubcoreMesh(core_axis_name="c", subcore_axis_name="s")

    @functools.partial(
        pl.kernel,
        mesh=mesh,
        out_type=jax.ShapeDtypeStruct((seq_len, batch, embed_dim), dtype),
        scratch_types=(
            [pltpu.VMEM((chunk, embed_dim), dtype) for _ in range(n_buf)]
            + [pltpu.SemaphoreType.DMA for _ in range(n_buf)]
            + [pltpu.SemaphoreType.DMA for _ in range(n_buf)]
        ),
    )
    def k(table_hbm, out_hbm, *scratch):
        bufs = scratch[:n_buf]
        lsem = scratch[n_buf : 2 * n_buf]
        wsem = scratch[2 * n_buf : 3 * n_buf]
        wid = lax.axis_index("s") * _NC + lax.axis_index("c")
        base = wid * rows_per_w

        def load(c):
            s0 = base + c * chunk
            d = pltpu.make_async_copy(
                table_hbm.at[pl.ds(s0, chunk)], bufs[c % n_buf], lsem[c % n_buf]
            )
            d.start()
            return d

        def writes(c):
            s0 = base + c * chunk
            ds = []
            for b in range(batch):
                d = pltpu.make_async_copy(
                    bufs[c % n_buf],
                    out_hbm.at[pl.ds(s0, chunk), b],
                    wsem[c % n_buf],
                )
                d.start()
                ds.append(d)
            return ds

        all_w = []
        lds = {}
        for c in range(min(n_buf, n_chunks)):
            lds[c] = load(c)
        for c in range(n_chunks):
            lds.pop(c).wait()
            all_w.extend(writes(c))
            if c + n_buf < n_chunks:
                lds[c + n_buf] = load(c + n_buf)
        for d in all_w:
            d.wait()

    return k


def kernel(x, pos_embedding):
    seq_len, batch = x.shape
    max_len, embed_dim = pos_embedding.shape
    k = _make_sc(seq_len, batch, embed_dim, pos_embedding.dtype)
    return k(pos_embedding)
